# Initial kernel scaffold; baseline (speedup 1.0000x reference)
import functools

import jax
import jax.numpy as jnp
from jax import lax
from jax.experimental import pallas as pl
from jax.experimental.pallas import tpu as pltpu
from jax.experimental.pallas import tpu_sc as plsc

NC = 2
NS = 16
NW = NC * NS

CH = 80
HCOLS = 8


def _fill_f32(ref, nvec, value):
    def body(i, _):
        ref[pl.ds(i * 16, 16)] = jnp.full((16,), value, jnp.float32)
        return 0
    lax.fori_loop(0, nvec, body, 0)


def _fill2d_f32(ref, rows, rowvecs, value):
    def body(i, _):
        r = i // rowvecs
        jj = i % rowvecs
        ref[r, pl.ds(jj * 16, 16)] = jnp.full((16,), value, jnp.float32)
        return 0
    lax.fori_loop(0, rows * rowvecs, body, 0)


def _sc_degree_kernel(npad, edges_per_tile):
    nchunk = edges_per_tile // CH
    zrows = npad // NS
    mesh = plsc.VectorSubcoreMesh(core_axis_name="c", subcore_axis_name="s")

    @functools.partial(
        pl.kernel,
        out_type=jax.ShapeDtypeStruct((NC, 2, npad, HCOLS), jnp.float32),
        mesh=mesh,
        scratch_types=[
            pltpu.VMEM((nchunk, CH), jnp.int32),
            pltpu.VMEM((CH, HCOLS), jnp.float32),
            pltpu.VMEM((zrows, HCOLS), jnp.float32),
            pltpu.VMEM_SHARED((npad, HCOLS), jnp.float32),
            pltpu.VMEM_SHARED((npad, HCOLS), jnp.float32),
        ],
    )
    def deg_kernel(edge_hbm, deg_hbm, idx_v, ones_v, zero_v, hist0, hist1):
        c = lax.axis_index("c")
        s = lax.axis_index("s")
        w = c * NS + s

        _fill2d_f32(ones_v, CH, HCOLS // 16 if HCOLS >= 16 else 0, 1.0)
        if HCOLS < 16:
            def obody(i, _):
                ones_v[pl.ds(i * 2, 2), :] = jnp.full((2, HCOLS), 1.0,
                                                      jnp.float32)
                return 0
            lax.fori_loop(0, CH // 2, obody, 0)
        _sc_zero2d(zero_v, zrows, HCOLS)
        pltpu.sync_copy(zero_v, hist0.at[pl.ds(s * zrows, zrows)])
        pltpu.sync_copy(zero_v, hist1.at[pl.ds(s * zrows, zrows)])
        plsc.subcore_barrier()

        for r, hist in ((0, hist0), (1, hist1)):
            pltpu.sync_copy(edge_hbm.at[r, pl.ds(w * nchunk, nchunk)], idx_v)

            def body(i, _):
                pltpu.sync_copy(ones_v, hist.at[idx_v.at[i]], add=True)
                return 0
            lax.fori_loop(0, nchunk, body, 0)

        plsc.subcore_barrier()
        pltpu.sync_copy(hist0.at[pl.ds(s * zrows, zrows)],
                        deg_hbm.at[c, 0, pl.ds(s * zrows, zrows)])
        pltpu.sync_copy(hist1.at[pl.ds(s * zrows, zrows)],
                        deg_hbm.at[c, 1, pl.ds(s * zrows, zrows)])

    return deg_kernel


def _sc_zero2d(ref, rows, cols):
    if cols >= 16:
        _fill2d_f32(ref, rows, cols // 16, 0.0)
    else:
        rpv = 16 // cols

        def body(i, _):
            ref[pl.ds(i * rpv, rpv), :] = jnp.zeros((rpv, cols), jnp.float32)
            return 0
        lax.fori_loop(0, rows // rpv, body, 0)


def _sc_aggregate_kernel(n_nodes, feat, edges_per_tile):
    nchunk = edges_per_tile // CH
    arows = n_nodes // NS
    zrows = 125
    mesh = plsc.VectorSubcoreMesh(core_axis_name="c", subcore_axis_name="s")

    @functools.partial(
        pl.kernel,
        out_type=jax.ShapeDtypeStruct((NC, n_nodes, feat), jnp.float32),
        mesh=mesh,
        scratch_types=[
            pltpu.VMEM((nchunk, CH), jnp.int32),
            pltpu.VMEM((nchunk, CH), jnp.int32),
            pltpu.VMEM((CH, feat), jnp.float32),
            pltpu.VMEM((125, feat), jnp.float32),
            pltpu.VMEM_SHARED((n_nodes, feat), jnp.float32),
            pltpu.SemaphoreType.DMA,
        ],
    )
    def agg_kernel(xs_hbm, edge_hbm, out_hbm,
                   src_v, dst_v, rows_v, zero_v, agg_sh, sem):
        c = lax.axis_index("c")
        s = lax.axis_index("s")
        w = c * NS + s

        _fill2d_f32(zero_v, zrows, feat // 16, 0.0)
        for j in range(arows // zrows):
            pltpu.sync_copy(
                zero_v, agg_sh.at[pl.ds(s * arows + j * zrows, zrows)])
        plsc.subcore_barrier()

        pltpu.sync_copy(edge_hbm.at[0, pl.ds(w * nchunk, nchunk)], src_v)
        pltpu.sync_copy(edge_hbm.at[1, pl.ds(w * nchunk, nchunk)], dst_v)

        def body(i, _):
            pltpu.async_copy(xs_hbm.at[src_v.at[i]], rows_v, sem).wait()
            pltpu.sync_copy(rows_v, agg_sh.at[dst_v.at[i]], add=True)
            return 0
        lax.fori_loop(0, nchunk, body, 0)

        plsc.subcore_barrier()
        pltpu.sync_copy(agg_sh.at[pl.ds(s * arows, arows)],
                        out_hbm.at[c, pl.ds(s * arows, arows)])

    return agg_kernel


def _tc_scale_transpose(x_ref, d_ref, xs_ref):
    x = x_ref[0]
    od = d_ref[0, 0, 0, :] + d_ref[1, 0, 0, :]
    ois = lax.rsqrt(jnp.maximum(od, 1.0))
    xsc = x * ois[None, :]
    f = x.shape[0]
    eye = (lax.broadcasted_iota(jnp.int32, (f, f), 0)
           == lax.broadcasted_iota(jnp.int32, (f, f), 1)).astype(jnp.float32)
    xs_ref[0] = lax.dot_general(xsc, eye, (((0,), (0,)), ((), ())),
                                preferred_element_type=jnp.float32)


def _tc_output(p_ref, d_ref, w_ref, b_ref, o_ref):
    agg = p_ref[0, 0] + p_ref[1, 0]
    iis = lax.rsqrt(jnp.maximum(d_ref[0, 1, 0, :] + d_ref[1, 1, 0, :], 1.0))
    ht = lax.dot_general(w_ref[...], agg, (((0,), (1,)), ((), ())),
                         preferred_element_type=jnp.float32)
    ht = ht * iis[None, :] + b_ref[...]
    o_ref[0] = jnp.where(ht >= 0, ht, 0.01 * ht)


def kernel(inputs, edge_index, W, b):
    sli, feat, node = inputs.shape
    n_nodes = sli * node
    E = edge_index.shape[1]
    npad = 10240
    edges_per_tile = E // NW

    edge3d = edge_index.reshape(2, E // CH, CH)

    deg_part = _sc_degree_kernel(npad, edges_per_tile)(edge3d)
    degs = deg_part[:, :, :n_nodes, 0].reshape(NC, 2, sli, node)

    xs = pl.pallas_call(
        _tc_scale_transpose,
        grid=(sli,),
        in_specs=[
            pl.BlockSpec((1, feat, node), lambda s: (s, 0, 0)),
            pl.BlockSpec((NC, 2, 1, node), lambda s: (0, 0, s, 0)),
        ],
        out_specs=pl.BlockSpec((1, node, feat), lambda s: (s, 0, 0)),
        out_shape=jax.ShapeDtypeStruct((sli, node, feat), jnp.float32),
    )(inputs, degs)
    xs = xs.reshape(n_nodes, feat)

    partials = _sc_aggregate_kernel(n_nodes, feat, edges_per_tile)(xs, edge3d)
    partials = partials.reshape(NC, sli, node, feat)

    out = pl.pallas_call(
        _tc_output,
        grid=(sli,),
        in_specs=[
            pl.BlockSpec((NC, 1, node, feat), lambda s: (0, s, 0, 0)),
            pl.BlockSpec((NC, 2, 1, node), lambda s: (0, 0, s, 0)),
            pl.BlockSpec((feat, feat), lambda s: (0, 0)),
            pl.BlockSpec((feat, 1), lambda s: (0, 0)),
        ],
        out_specs=pl.BlockSpec((1, feat, node), lambda s: (s, 0, 0)),
        out_shape=jax.ShapeDtypeStruct((sli, feat, node), jnp.float32),
    )(partials, degs, W, b.reshape(feat, 1))
    return out


# speedup vs baseline: 5.8358x; 5.8358x over previous
import functools

import jax
import jax.numpy as jnp
from jax import lax
from jax.experimental import pallas as pl
from jax.experimental.pallas import tpu as pltpu
from jax.experimental.pallas import tpu_sc as plsc

NC = 2
NS = 16
NW = NC * NS

CH = 80
HCOLS = 128


def _sc_degree_kernel(npad, edges_per_tile):
    nchunk = edges_per_tile // CH
    zrows = npad // NS
    mesh = plsc.VectorSubcoreMesh(core_axis_name="c", subcore_axis_name="s")

    @functools.partial(
        pl.kernel,
        out_type=jax.ShapeDtypeStruct((NC, 2, NS, zrows, HCOLS), jnp.float32),
        mesh=mesh,
        scratch_types=[
            pltpu.VMEM((nchunk, CH), jnp.int32),
            pltpu.VMEM((CH, HCOLS), jnp.float32),
            pltpu.VMEM_SHARED((npad, HCOLS), jnp.float32),
        ],
    )
    def deg_kernel(edge_hbm, ones_hbm, zeros_hbm, deg_hbm,
                   idx_v, ones_v, hist):
        c = lax.axis_index("c")
        s = lax.axis_index("s")
        w = c * NS + s

        pltpu.sync_copy(ones_hbm, ones_v)
        for r in (0, 1):
            pltpu.sync_copy(zeros_hbm, hist.at[pl.ds(s * zrows, zrows)])
            pltpu.sync_copy(edge_hbm.at[r, w], idx_v)
            plsc.subcore_barrier()

            def body(i, _):
                pltpu.sync_copy(ones_v, hist.at[idx_v.at[i]], add=True)
                return 0
            lax.fori_loop(0, nchunk, body, 0)

            plsc.subcore_barrier()
            pltpu.sync_copy(hist.at[pl.ds(s * zrows, zrows)],
                            deg_hbm.at[c, r, s])

    return deg_kernel


def _sc_aggregate_kernel(n_nodes, feat, edges_per_tile):
    nchunk = edges_per_tile // CH
    arows = n_nodes // NS
    mesh = plsc.VectorSubcoreMesh(core_axis_name="c", subcore_axis_name="s")

    @functools.partial(
        pl.kernel,
        out_type=jax.ShapeDtypeStruct((NC, NS, n_nodes // NS, feat),
                                      jnp.float32),
        mesh=mesh,
        scratch_types=[
            pltpu.VMEM((nchunk, CH), jnp.int32),
            pltpu.VMEM((nchunk, CH), jnp.int32),
            pltpu.VMEM((CH, feat), jnp.float32),
            pltpu.VMEM_SHARED((n_nodes, feat), jnp.float32),
            pltpu.SemaphoreType.DMA,
        ],
    )
    def agg_kernel(xs_hbm, edge_hbm, zeros_hbm, out_hbm,
                   src_v, dst_v, rows_v, agg_sh, sem):
        c = lax.axis_index("c")
        s = lax.axis_index("s")
        w = c * NS + s

        pltpu.sync_copy(zeros_hbm, agg_sh.at[pl.ds(s * arows, arows)])
        plsc.subcore_barrier()

        pltpu.sync_copy(edge_hbm.at[0, w], src_v)
        pltpu.sync_copy(edge_hbm.at[1, w], dst_v)

        def body(i, _):
            pltpu.async_copy(xs_hbm.at[src_v.at[i]], rows_v, sem).wait()
            pltpu.sync_copy(rows_v, agg_sh.at[dst_v.at[i]], add=True)
            return 0
        lax.fori_loop(0, nchunk, body, 0)

        plsc.subcore_barrier()
        pltpu.sync_copy(agg_sh.at[pl.ds(s * arows, arows)], out_hbm.at[c, s])

    return agg_kernel


def _tc_scale_transpose(x_ref, d_ref, xs_ref):
    x = x_ref[0]
    od = d_ref[0, 0, 0, :] + d_ref[0, 1, 0, :]
    ois = lax.rsqrt(jnp.maximum(od, 1.0))
    xsc = x * ois[None, :]
    f = x.shape[0]
    eye = (lax.broadcasted_iota(jnp.int32, (f, f), 0)
           == lax.broadcasted_iota(jnp.int32, (f, f), 1)).astype(jnp.float32)
    xs_ref[0] = lax.dot_general(xsc, eye, (((0,), (0,)), ((), ())),
                                preferred_element_type=jnp.float32)


def _tc_output(p_ref, d_ref, w_ref, b_ref, o_ref):
    agg = p_ref[0, 0] + p_ref[1, 0]
    iis = lax.rsqrt(jnp.maximum(d_ref[0, 0, 1, :] + d_ref[0, 1, 1, :], 1.0))
    ht = lax.dot_general(w_ref[...], agg, (((0,), (1,)), ((), ())),
                         preferred_element_type=jnp.float32)
    ht = ht * iis[None, :] + b_ref[...]
    o_ref[0] = jnp.where(ht >= 0, ht, 0.01 * ht)


def kernel(inputs, edge_index, W, b):
    sli, feat, node = inputs.shape
    n_nodes = sli * node
    E = edge_index.shape[1]
    npad = 10240
    edges_per_tile = E // NW

    nchunk = edges_per_tile // CH
    edge4d = edge_index.reshape(2, NW, nchunk, CH)
    ones_rows = jnp.ones((CH, HCOLS), jnp.float32)
    zeros_hist = jnp.zeros((npad // NS, HCOLS), jnp.float32)
    zeros_agg = jnp.zeros((n_nodes // NS, feat), jnp.float32)

    deg_fn = _sc_degree_kernel(npad, edges_per_tile)
    deg_part = deg_fn(edge4d, ones_rows, zeros_hist)
    degs = deg_part.reshape(NC, 2, npad, HCOLS)[:, :, :n_nodes, 0]
    degs = jnp.transpose(degs.reshape(NC, 2, sli, node), (2, 0, 1, 3))

    xs = pl.pallas_call(
        _tc_scale_transpose,
        grid=(sli,),
        in_specs=[
            pl.BlockSpec((1, feat, node), lambda s: (s, 0, 0)),
            pl.BlockSpec((1, NC, 2, node), lambda s: (s, 0, 0, 0)),
        ],
        out_specs=pl.BlockSpec((1, node, feat), lambda s: (s, 0, 0)),
        out_shape=jax.ShapeDtypeStruct((sli, node, feat), jnp.float32),
    )(inputs, degs)
    xs = xs.reshape(n_nodes, feat)

    agg_fn = _sc_aggregate_kernel(n_nodes, feat, edges_per_tile)
    partials = agg_fn(xs, edge4d, zeros_agg)
    partials = partials.reshape(NC, sli, node, feat)

    out = pl.pallas_call(
        _tc_output,
        grid=(sli,),
        in_specs=[
            pl.BlockSpec((NC, 1, node, feat), lambda s: (0, s, 0, 0)),
            pl.BlockSpec((1, NC, 2, node), lambda s: (s, 0, 0, 0)),
            pl.BlockSpec((feat, feat), lambda s: (0, 0)),
            pl.BlockSpec((feat, 1), lambda s: (0, 0)),
        ],
        out_specs=pl.BlockSpec((1, feat, node), lambda s: (s, 0, 0)),
        out_shape=jax.ShapeDtypeStruct((sli, feat, node), jnp.float32),
    )(partials, degs, W, b.reshape(feat, 1))
    return out
